# baseline (speedup 1.0000x reference)
import jax
import jax.numpy as jnp
from jax import lax
from jax.experimental import pallas as pl

_TOP_K = 3
_EPS = 2.220446049250313e-16
_NEG_INF = float("-inf")


def _beam_kernel(x_ref, tok_ref, sc_ref):
    B, L, V = x_ref.shape
    K = _TOP_K

    x = x_ref[...]
    m = jnp.max(x, axis=2, keepdims=True)
    e = jnp.exp(x - m)
    s = jnp.sum(e, axis=2, keepdims=True)
    lp = jnp.log(e / s + _EPS)

    iota_v = lax.broadcasted_iota(jnp.int32, (B, L, V), 2)
    vals, toks = [], []
    work = lp
    for i in range(K):
        v = jnp.max(work, axis=2, keepdims=True)
        idx = jnp.min(jnp.where(work == v, iota_v, V), axis=2, keepdims=True)
        vals.append(v)
        toks.append(idx)
        if i + 1 < K:
            work = jnp.where(iota_v == idx, _NEG_INF, work)

    def at_t(a, t):
        return a[:, t, :]

    def argmax9(c):
        ent = [(c[q], q) for q in range(9)]
        while len(ent) > 1:
            nxt = []
            for a in range(0, len(ent) - 1, 2):
                (va, qa), (vb, qb) = ent[a], ent[a + 1]
                take_a = va >= vb
                qa_arr = jnp.asarray(qa, jnp.int32)
                qb_arr = jnp.asarray(qb, jnp.int32)
                nxt.append((jnp.maximum(va, vb),
                            jnp.where(take_a, qa_arr, qb_arr)))
            if len(ent) % 2:
                nxt.append(ent[-1])
            ent = nxt
        return ent[0]

    scores = [at_t(vals[i], 0) for i in range(K)]
    iota_c = lax.broadcasted_iota(jnp.int32, (B, L), 1)
    seqs = [jnp.where(iota_c == 0, at_t(toks[i], 0), 0) for i in range(K)]

    for t in range(1, L):
        vt = [at_t(vals[i], t) for i in range(K)]
        tt = [at_t(toks[i], t) for i in range(K)]
        c = [scores[k] + vt[i] for k in range(K) for i in range(K)]
        new_scores, new_seqs = [], []
        for _j in range(K):
            best, sel = argmax9(c)
            ge3 = (sel >= K).astype(jnp.int32)
            ge6 = (sel >= 2 * K).astype(jnp.int32)
            beam = ge3 + ge6
            ipick = sel - K * beam
            tok = jnp.where(ipick == 0, tt[0],
                            jnp.where(ipick == 1, tt[1], tt[2]))
            g = jnp.where(beam == 0, seqs[0],
                          jnp.where(beam == 1, seqs[1], seqs[2]))
            g = jnp.where(iota_c == t, tok, g)
            new_scores.append(best)
            new_seqs.append(g)
            c = [jnp.where(sel == q, _NEG_INF, c[q]) for q in range(9)]
        scores, seqs = new_scores, new_seqs

    tok_ref[...] = jnp.stack(seqs, axis=-1).astype(jnp.int32)
    sc_ref[...] = jnp.concatenate(scores, axis=1)


def kernel(logits):
    B, L, V = logits.shape
    return pl.pallas_call(
        _beam_kernel,
        out_shape=(
            jax.ShapeDtypeStruct((B, L, _TOP_K), jnp.int32),
            jax.ShapeDtypeStruct((B, _TOP_K), jnp.float32),
        ),
    )(logits)


# speedup vs baseline: 1.0963x; 1.0963x over previous
import jax
import jax.numpy as jnp
from jax import lax
from jax.experimental import pallas as pl

_TOP_K = 3
_EPS = 2.220446049250313e-16
_NEG_INF = float("-inf")


def _beam_kernel(x_ref, tok_ref, sc_ref):
    B, L, V = x_ref.shape
    K = _TOP_K

    x = x_ref[...]
    m = jnp.max(x, axis=2, keepdims=True)
    e = jnp.exp(x - m)
    s = jnp.sum(e, axis=2, keepdims=True)
    lp = jnp.log(e / s + _EPS)

    iota_v = lax.broadcasted_iota(jnp.int32, (B, V), 1)
    vals = [[None] * K for _ in range(L)]
    toks = [[None] * K for _ in range(L)]
    for t in range(L):
        work = lp[:, t, :]
        for i in range(K):
            v = jnp.max(work, axis=1, keepdims=True)
            idx = jnp.min(jnp.where(work == v, iota_v, V),
                          axis=1, keepdims=True)
            vals[t][i] = v
            toks[t][i] = idx
            if i + 1 < K:
                work = jnp.where(iota_v == idx, _NEG_INF, work)

    def argmax9(c):
        ent = [(c[q], q) for q in range(9)]
        while len(ent) > 1:
            nxt = []
            for a in range(0, len(ent) - 1, 2):
                (va, qa), (vb, qb) = ent[a], ent[a + 1]
                take_a = va >= vb
                qa_arr = jnp.asarray(qa, jnp.int32)
                qb_arr = jnp.asarray(qb, jnp.int32)
                nxt.append((jnp.maximum(va, vb),
                            jnp.where(take_a, qa_arr, qb_arr)))
            if len(ent) % 2:
                nxt.append(ent[-1])
            ent = nxt
        return ent[0]

    scores = [vals[0][i] for i in range(K)]
    iota_c = lax.broadcasted_iota(jnp.int32, (B, L), 1)
    seqs = [jnp.where(iota_c == 0, toks[0][i], 0) for i in range(K)]

    for t in range(1, L):
        vt = vals[t]
        tt = toks[t]
        c = [scores[k] + vt[i] for k in range(K) for i in range(K)]
        new_scores, new_seqs = [], []
        for _j in range(K):
            best, sel = argmax9(c)
            ge3 = (sel >= K).astype(jnp.int32)
            ge6 = (sel >= 2 * K).astype(jnp.int32)
            beam = ge3 + ge6
            ipick = sel - K * beam
            tok = jnp.where(ipick == 0, tt[0],
                            jnp.where(ipick == 1, tt[1], tt[2]))
            g = jnp.where(beam == 0, seqs[0],
                          jnp.where(beam == 1, seqs[1], seqs[2]))
            g = jnp.where(iota_c == t, tok, g)
            new_scores.append(best)
            new_seqs.append(g)
            c = [jnp.where(sel == q, _NEG_INF, c[q]) for q in range(9)]
        scores, seqs = new_scores, new_seqs

    tok_ref[...] = jnp.stack(seqs, axis=-1).astype(jnp.int32)
    sc_ref[...] = jnp.concatenate(scores, axis=1)


def kernel(logits):
    B, L, V = logits.shape
    return pl.pallas_call(
        _beam_kernel,
        out_shape=(
            jax.ShapeDtypeStruct((B, L, _TOP_K), jnp.int32),
            jax.ShapeDtypeStruct((B, _TOP_K), jnp.float32),
        ),
    )(logits)


# speedup vs baseline: 1.1139x; 1.0161x over previous
import jax
import jax.numpy as jnp
from jax import lax
from jax.experimental import pallas as pl

_TOP_K = 3
_EPS = 2.220446049250313e-16
_NEG_INF = float("-inf")


def _beam_kernel(x_ref, tok_ref, sc_ref):
    B, L, V = x_ref.shape
    K = _TOP_K

    iota_v = lax.broadcasted_iota(jnp.int32, (B, V), 1)
    vals = [[None] * K for _ in range(L)]
    toks = [[None] * K for _ in range(L)]
    for t in range(L):
        x = x_ref[:, t, :]
        m = jnp.max(x, axis=1, keepdims=True)
        e = jnp.exp(x - m)
        s = jnp.sum(e, axis=1, keepdims=True)
        work = jnp.log(e / s + _EPS)
        for i in range(K):
            v = jnp.max(work, axis=1, keepdims=True)
            idx = jnp.min(jnp.where(work == v, iota_v, V),
                          axis=1, keepdims=True)
            vals[t][i] = v
            toks[t][i] = idx
            if i + 1 < K:
                work = jnp.where(iota_v == idx, _NEG_INF, work)

    def argmax9(c):
        ent = [(c[q], q) for q in range(9)]
        while len(ent) > 1:
            nxt = []
            for a in range(0, len(ent) - 1, 2):
                (va, qa), (vb, qb) = ent[a], ent[a + 1]
                take_a = va >= vb
                qa_arr = jnp.asarray(qa, jnp.int32)
                qb_arr = jnp.asarray(qb, jnp.int32)
                nxt.append((jnp.maximum(va, vb),
                            jnp.where(take_a, qa_arr, qb_arr)))
            if len(ent) % 2:
                nxt.append(ent[-1])
            ent = nxt
        return ent[0]

    scores = [vals[0][i] for i in range(K)]
    iota_c = lax.broadcasted_iota(jnp.int32, (B, L), 1)
    seqs = [jnp.where(iota_c == 0, toks[0][i], 0) for i in range(K)]

    for t in range(1, L):
        vt = vals[t]
        tt = toks[t]
        c = [scores[k] + vt[i] for k in range(K) for i in range(K)]
        new_scores, new_seqs = [], []
        for _j in range(K):
            best, sel = argmax9(c)
            ge3 = (sel >= K).astype(jnp.int32)
            ge6 = (sel >= 2 * K).astype(jnp.int32)
            beam = ge3 + ge6
            ipick = sel - K * beam
            tok = jnp.where(ipick == 0, tt[0],
                            jnp.where(ipick == 1, tt[1], tt[2]))
            g = jnp.where(beam == 0, seqs[0],
                          jnp.where(beam == 1, seqs[1], seqs[2]))
            g = jnp.where(iota_c == t, tok, g)
            new_scores.append(best)
            new_seqs.append(g)
            c = [jnp.where(sel == q, _NEG_INF, c[q]) for q in range(9)]
        scores, seqs = new_scores, new_seqs

    tok_ref[...] = jnp.stack(seqs, axis=-1).astype(jnp.int32)
    sc_ref[...] = jnp.concatenate(scores, axis=1)


def kernel(logits):
    B, L, V = logits.shape
    return pl.pallas_call(
        _beam_kernel,
        out_shape=(
            jax.ShapeDtypeStruct((B, L, _TOP_K), jnp.int32),
            jax.ShapeDtypeStruct((B, _TOP_K), jnp.float32),
        ),
    )(logits)
